# baseline (speedup 1.0000x reference)
import functools

import jax
import jax.numpy as jnp
from jax import lax
from jax.experimental import pallas as pl
from jax.experimental.pallas import tpu as pltpu
from jax.experimental.pallas import tpu_sc as plsc

_INFO = plsc.get_sparse_core_info()
_NC = _INFO.num_cores
_NS = _INFO.num_subcores
_NW = _NC * _NS
_L = 16


@jax.jit
def _embedding_lookup(tableq, idx_t):
    S, Bt = idx_t.shape
    Q, W = tableq.shape
    D = 32
    PK = W // D
    seq_per_w = Bt // _NW
    CH = D // 8
    NB = 4
    mesh = plsc.VectorSubcoreMesh(core_axis_name="c", subcore_axis_name="s")

    @functools.partial(
        pl.kernel,
        out_type=jax.ShapeDtypeStruct((S, CH, _NW, 8, seq_per_w),
                                      jnp.float32),
        mesh=mesh,
        compiler_params=pltpu.CompilerParams(use_tc_tiling_on_sc=False,
                                             needs_layout_passes=False),
        scratch_types=[
            pltpu.VMEM((S, seq_per_w), jnp.int32),
            pltpu.VMEM((NB, seq_per_w), jnp.int32),
            pltpu.VMEM((NB, seq_per_w, W), jnp.float32),
            pltpu.VMEM((2, CH, 8, seq_per_w), jnp.float32),
            pltpu.SemaphoreType.DMA((NB,)),
            pltpu.SemaphoreType.DMA((2,)),
        ],
    )
    def emb(tab_hbm, idx_hbm, out_hbm, idx_v, q_v, rows_v, t_v, gsem, wsem):
        wid = lax.axis_index("s") * _NC + lax.axis_index("c")
        base = wid * seq_per_w
        iota = lax.iota(jnp.int32, _L)

        def prep_gather(p, b):
            @pl.loop(0, seq_per_w // _L)
            def _(k):
                v = idx_v[p, pl.ds(k * _L, _L)]
                q_v[b, pl.ds(k * _L, _L)] = lax.shift_right_logical(
                    v, jnp.int32(2))
            pltpu.async_copy(tab_hbm.at[q_v.at[b]], rows_v.at[b], gsem.at[b])

        def wait_gather(b):
            pltpu.make_async_copy(tab_hbm.at[q_v.at[b]], rows_v.at[b],
                                  gsem.at[b]).wait()

        def transpose(p, b, tb):
            @pl.loop(0, seq_per_w // _L)
            def _(k):
                si = iota + k * _L
                sub = lax.shift_left(
                    lax.bitwise_and(idx_v[p, pl.ds(k * _L, _L)],
                                    jnp.int32(3)), jnp.int32(5))
                for ch in range(CH):
                    for ci in range(8):
                        col = sub + jnp.int32(ch * 8 + ci)
                        v = plsc.load_gather(rows_v.at[b], [si, col])
                        t_v[tb, ch, ci, pl.ds(k * _L, _L)] = v

        def start_wb(p, tb):
            pltpu.async_copy(t_v.at[tb], out_hbm.at[p, :, wid], wsem.at[tb])

        def wait_wb(p, tb):
            pltpu.make_async_copy(t_v.at[tb], out_hbm.at[p, :, wid],
                                  wsem.at[tb]).wait()

        pltpu.sync_copy(idx_hbm.at[:, pl.ds(base, seq_per_w)], idx_v)

        for b in range(NB):
            prep_gather(b, b)

        for b in range(NB):
            wait_gather(b)
            if b >= 2:
                wait_wb(b - 2, b % 2)
            transpose(b, b, b % 2)
            start_wb(b, b % 2)
            prep_gather(b + NB, b)

        @pl.loop(NB, S - NB, step=NB)
        def ring(g):
            for b in range(NB):
                p = g + b
                wait_gather(b)
                wait_wb(p - 2, b % 2)
                transpose(p, b, b % 2)
                start_wb(p, b % 2)
                prep_gather(p + NB, b)

        for b in range(NB):
            p = S - NB + b
            wait_gather(b)
            wait_wb(p - 2, b % 2)
            transpose(p, b, b % 2)
            start_wb(p, b % 2)
        wait_wb(S - 2, 0)
        wait_wb(S - 1, 1)

    return emb(tableq, idx_t)


def kernel(token_indices, embedding_table):
    Bt, S = token_indices.shape
    V, D = embedding_table.shape
    tableq = embedding_table.reshape(V // 4, 4 * D)
    out5 = _embedding_lookup(tableq, token_indices.T.astype(jnp.int32))
    return out5.transpose(2, 4, 0, 1, 3).reshape(Bt, S, D)


# speedup vs baseline: 1.0159x; 1.0159x over previous
import functools

import jax
import jax.numpy as jnp
from jax import lax
from jax.experimental import pallas as pl
from jax.experimental.pallas import tpu as pltpu
from jax.experimental.pallas import tpu_sc as plsc

_INFO = plsc.get_sparse_core_info()
_NC = _INFO.num_cores
_NS = _INFO.num_subcores
_NW = _NC * _NS
_L = 16


@jax.jit
def _embedding_lookup(table, idx_t):
    S, Bt = idx_t.shape
    V, D = table.shape
    seq_per_w = Bt // _NW
    CH = D // 8
    NB = 4
    mesh = plsc.VectorSubcoreMesh(core_axis_name="c", subcore_axis_name="s")

    @functools.partial(
        pl.kernel,
        out_type=jax.ShapeDtypeStruct((S, CH, _NW, 8, seq_per_w),
                                      jnp.float32),
        mesh=mesh,
        compiler_params=pltpu.CompilerParams(use_tc_tiling_on_sc=False,
                                             needs_layout_passes=False),
        scratch_types=[
            pltpu.VMEM((S, seq_per_w), jnp.int32),
            pltpu.VMEM((NB, seq_per_w, D), jnp.float32),
            pltpu.VMEM((2, CH, 8, seq_per_w), jnp.float32),
            pltpu.SemaphoreType.DMA((NB,)),
            pltpu.SemaphoreType.DMA((2,)),
        ],
    )
    def emb(tab_hbm, idx_hbm, out_hbm, idx_v, rows_v, t_v, gsem, wsem):
        wid = lax.axis_index("s") * _NC + lax.axis_index("c")
        base = wid * seq_per_w
        iota = lax.iota(jnp.int32, _L)

        def start_gather(p, b):
            pltpu.async_copy(tab_hbm.at[idx_v.at[p]], rows_v.at[b],
                             gsem.at[b])

        def wait_gather(p, b):
            pltpu.make_async_copy(tab_hbm.at[idx_v.at[p]], rows_v.at[b],
                                  gsem.at[b]).wait()

        def transpose(b, tb):
            @pl.loop(0, seq_per_w // _L)
            def _(k):
                si = iota + k * _L
                for ch in range(CH):
                    for ci in range(8):
                        col = lax.broadcast(jnp.int32(ch * 8 + ci), (_L,))
                        v = plsc.load_gather(rows_v.at[b], [si, col])
                        t_v[tb, ch, ci, pl.ds(k * _L, _L)] = v

        def start_wb(p, tb):
            pltpu.async_copy(t_v.at[tb], out_hbm.at[p, :, wid], wsem.at[tb])

        def wait_wb(p, tb):
            pltpu.make_async_copy(t_v.at[tb], out_hbm.at[p, :, wid],
                                  wsem.at[tb]).wait()

        pltpu.sync_copy(idx_hbm.at[:, pl.ds(base, seq_per_w)], idx_v)

        for b in range(NB):
            start_gather(b, b)

        for b in range(NB):
            wait_gather(b, b)
            if b >= 2:
                wait_wb(b - 2, b % 2)
            transpose(b, b % 2)
            start_wb(b, b % 2)
            start_gather(b + NB, b)

        @pl.loop(NB, S - NB, step=NB)
        def ring(g):
            for b in range(NB):
                p = g + b
                wait_gather(p, b)
                wait_wb(p - 2, b % 2)
                transpose(b, b % 2)
                start_wb(p, b % 2)
                start_gather(p + NB, b)

        for b in range(NB):
            p = S - NB + b
            wait_gather(p, b)
            wait_wb(p - 2, b % 2)
            transpose(b, b % 2)
            start_wb(p, b % 2)
        wait_wb(S - 2, 0)
        wait_wb(S - 1, 1)

    return emb(table, idx_t)


def kernel(token_indices, embedding_table):
    Bt, S = token_indices.shape
    V, D = embedding_table.shape
    out5 = _embedding_lookup(embedding_table,
                             token_indices.T.astype(jnp.int32))
    return out5.transpose(2, 4, 0, 1, 3).reshape(Bt, S, D)


# speedup vs baseline: 1.5506x; 1.5263x over previous
import functools

import jax
import jax.numpy as jnp
from jax import lax
from jax.experimental import pallas as pl
from jax.experimental.pallas import tpu as pltpu
from jax.experimental.pallas import tpu_sc as plsc

_INFO = plsc.get_sparse_core_info()
_NC = _INFO.num_cores
_NS = _INFO.num_subcores
_NW = _NC * _NS
_L = 16


@jax.jit
def _embedding_lookup(table, idx_t):
    S, Bt = idx_t.shape
    V, D = table.shape
    seq_per_w = Bt // _NW
    CH = D // 8
    NB = 4
    mesh = plsc.VectorSubcoreMesh(core_axis_name="c", subcore_axis_name="s")

    @functools.partial(
        pl.kernel,
        out_type=jax.ShapeDtypeStruct((S, CH, _NW, 8, seq_per_w),
                                      jnp.float32),
        mesh=mesh,
        compiler_params=pltpu.CompilerParams(use_tc_tiling_on_sc=False,
                                             needs_layout_passes=False),
        scratch_types=[
            pltpu.VMEM((S, seq_per_w), jnp.int32),
            pltpu.VMEM((NB, seq_per_w, D), jnp.float32),
            pltpu.VMEM((2, CH, 8, seq_per_w + 1), jnp.float32),
            pltpu.SemaphoreType.DMA((NB,)),
            pltpu.SemaphoreType.DMA((2,)),
        ],
    )
    def emb(tab_hbm, idx_hbm, out_hbm, idx_v, rows_v, t_v, gsem, wsem):
        wid = lax.axis_index("s") * _NC + lax.axis_index("c")
        base = wid * seq_per_w
        iota = lax.iota(jnp.int32, _L)

        def start_gather(p, b):
            pltpu.async_copy(tab_hbm.at[idx_v.at[p]], rows_v.at[b],
                             gsem.at[b])

        def wait_gather(p, b):
            pltpu.make_async_copy(tab_hbm.at[idx_v.at[p]], rows_v.at[b],
                                  gsem.at[b]).wait()

        tb_idx = lax.broadcast(jnp.int32(0), (_L,))
        ch_lo = lax.shift_right_logical(iota, jnp.int32(3))
        ci_lo = lax.bitwise_and(iota, jnp.int32(7))

        def transpose(b, tb):
            tbv = tb_idx + jnp.int32(tb)
            @pl.loop(0, seq_per_w // _L)
            def _(k):
                for j in range(_L):
                    si = k * _L + j
                    siv = lax.broadcast(si, (_L,))
                    for h in range(D // _L):
                        v = rows_v[b, si, pl.ds(h * _L, _L)]
                        plsc.store_scatter(
                            t_v,
                            [tbv, ch_lo + jnp.int32(h * 2), ci_lo, siv], v)

        def start_wb(p, tb):
            pltpu.async_copy(t_v.at[tb, :, :, pl.ds(0, seq_per_w)],
                             out_hbm.at[p, :, wid], wsem.at[tb])

        def wait_wb(p, tb):
            pltpu.make_async_copy(t_v.at[tb, :, :, pl.ds(0, seq_per_w)],
                                  out_hbm.at[p, :, wid], wsem.at[tb]).wait()

        pltpu.sync_copy(idx_hbm.at[:, pl.ds(base, seq_per_w)], idx_v)

        for b in range(NB):
            start_gather(b, b)

        for b in range(NB):
            wait_gather(b, b)
            if b >= 2:
                wait_wb(b - 2, b % 2)
            transpose(b, b % 2)
            start_wb(b, b % 2)
            start_gather(b + NB, b)

        @pl.loop(NB, S - NB, step=NB)
        def ring(g):
            for b in range(NB):
                p = g + b
                wait_gather(p, b)
                wait_wb(p - 2, b % 2)
                transpose(b, b % 2)
                start_wb(p, b % 2)
                start_gather(p + NB, b)

        for b in range(NB):
            p = S - NB + b
            wait_gather(p, b)
            wait_wb(p - 2, b % 2)
            transpose(b, b % 2)
            start_wb(p, b % 2)
        wait_wb(S - 2, 0)
        wait_wb(S - 1, 1)

    return emb(table, idx_t)


def kernel(token_indices, embedding_table):
    Bt, S = token_indices.shape
    V, D = embedding_table.shape
    out5 = _embedding_lookup(embedding_table,
                             token_indices.T.astype(jnp.int32))
    return out5.transpose(2, 4, 0, 1, 3).reshape(Bt, S, D)
